# baseline (speedup 1.0000x reference)
import functools

import jax
import jax.numpy as jnp
from jax import lax
from jax.experimental import pallas as pl
from jax.experimental.pallas import tpu as pltpu
from jax.experimental.pallas import tpu_sc as plsc

N = 50048
E = 800768
NROWS = E // 128
HID = 64
NSLICE = N // 16

_MESH = plsc.VectorSubcoreMesh(core_axis_name="c", subcore_axis_name="s")

_f32 = jnp.float32
_i32 = jnp.int32


def _wid():
    return lax.axis_index("s") * 2 + lax.axis_index("c")


def _chunk_range(wid):
    lo = wid * 195 + jnp.minimum(wid, 16)
    cnt = jnp.where(wid < 16, 196, 195)
    return lo, cnt


_IOTA16 = lambda: lax.broadcasted_iota(_i32, (16,), 0)


@functools.partial(
    pl.kernel,
    out_type=(jax.ShapeDtypeStruct((NROWS, 128), _f32),
              jax.ShapeDtypeStruct((32, N), _f32)),
    mesh=_MESH,
    compiler_params=pltpu.CompilerParams(needs_layout_passes=False, use_tc_tiling_on_sc=False),
    scratch_types=[
        pltpu.VMEM((128,), _i32),
        pltpu.VMEM((128,), _i32),
        pltpu.VMEM((128, HID), _f32),
        pltpu.VMEM((128, HID), _f32),
        pltpu.VMEM((128,), _f32),
        pltpu.VMEM((128,), _f32),
        pltpu.VMEM((N,), _f32),
        pltpu.SemaphoreType.DMA,
        pltpu.SemaphoreType.DMA,
    ],
)
def _k1(xl_hbm, xr_hbm, att_hbm, src_hbm, dst_hbm, zero1_hbm,
        w_out, s_out,
        src_iv, dst_iv, srows, drows, att_v, wbuf, s_part, sem1, sem2):
    wid = _wid()
    pltpu.sync_copy(att_hbm, att_v)
    pltpu.sync_copy(zero1_hbm, s_part)
    lo, cnt = _chunk_range(wid)
    iota16 = _IOTA16()

    def chunk(i, carry):
        row = lo + i
        pltpu.sync_copy(src_hbm.at[row], src_iv)
        pltpu.sync_copy(dst_hbm.at[row], dst_iv)
        cp1 = pltpu.async_copy(xl_hbm.at[src_iv], srows, sem1)
        cp2 = pltpu.async_copy(xr_hbm.at[dst_iv], drows, sem2)
        cp1.wait()
        cp2.wait()

        def dbody(d, accs):
            attb = plsc.load_gather(att_v, [iota16 * 0 + d])
            out = []
            for g in range(8):
                rid = iota16 + g * 16
                cid = iota16 * 0 + d
                s16 = plsc.load_gather(srows, [rid, cid])
                d16 = plsc.load_gather(drows, [rid, cid])
                za = s16 + d16
                lr = 0.6 * za + 0.4 * jnp.abs(za)
                out.append(accs[g] + lr * attb)
            return tuple(out)

        accs = lax.fori_loop(0, HID, dbody,
                             tuple(jnp.zeros((16,), _f32) for _ in range(8)))
        for g in range(8):
            w16 = jnp.exp(accs[g])
            wbuf[pl.ds(g * 16, 16)] = w16
            dst16 = dst_iv[pl.ds(g * 16, 16)]
            plsc.addupdate_scatter(s_part, [dst16], w16)
        pltpu.sync_copy(wbuf, w_out.at[row])
        return carry

    lax.fori_loop(0, cnt, chunk, 0)
    pltpu.sync_copy(s_part, s_out.at[wid])


@functools.partial(
    pl.kernel,
    out_type=jax.ShapeDtypeStruct((NROWS, 128), _f32),
    mesh=_MESH,
    compiler_params=pltpu.CompilerParams(needs_layout_passes=False, use_tc_tiling_on_sc=False),
    scratch_types=[
        pltpu.VMEM((128,), _i32),
        pltpu.VMEM((128,), _i32),
        pltpu.VMEM((N,), _f32),
        pltpu.VMEM((N,), _f32),
        pltpu.VMEM((128,), _f32),
    ],
)
def _k3(asrc_hbm, adst_hbm, src_hbm, dst_hbm,
        w_out,
        src_iv, dst_iv, asv, adv, wbuf):
    wid = _wid()
    pltpu.sync_copy(asrc_hbm, asv)
    pltpu.sync_copy(adst_hbm, adv)
    lo, cnt = _chunk_range(wid)

    def chunk(i, carry):
        row = lo + i
        pltpu.sync_copy(src_hbm.at[row], src_iv)
        pltpu.sync_copy(dst_hbm.at[row], dst_iv)
        for g in range(8):
            src16 = src_iv[pl.ds(g * 16, 16)]
            dst16 = dst_iv[pl.ds(g * 16, 16)]
            a16 = plsc.load_gather(asv, [src16])
            b16 = plsc.load_gather(adv, [dst16])
            z = a16 + b16
            lr = 0.6 * z + 0.4 * jnp.abs(z)
            wbuf[pl.ds(g * 16, 16)] = jnp.exp(lr)
        pltpu.sync_copy(wbuf, w_out.at[row])
        return carry

    lax.fori_loop(0, cnt, chunk, 0)


@functools.partial(
    pl.kernel,
    out_type=jax.ShapeDtypeStruct((32, N), _f32),
    mesh=_MESH,
    compiler_params=pltpu.CompilerParams(needs_layout_passes=False, use_tc_tiling_on_sc=False),
    scratch_types=[
        pltpu.VMEM((128,), _i32),
        pltpu.VMEM((128,), _f32),
        pltpu.VMEM((N,), _f32),
    ],
)
def _ks(dst_hbm, w_hbm, zero1_hbm,
        s_out,
        dst_iv, w_iv, s_part):
    wid = _wid()
    pltpu.sync_copy(zero1_hbm, s_part)
    lo, cnt = _chunk_range(wid)

    def chunk(i, carry):
        row = lo + i
        pltpu.sync_copy(dst_hbm.at[row], dst_iv)
        pltpu.sync_copy(w_hbm.at[row], w_iv)
        for g in range(8):
            dst16 = dst_iv[pl.ds(g * 16, 16)]
            w16 = w_iv[pl.ds(g * 16, 16)]
            plsc.addupdate_scatter(s_part, [dst16], w16)
        return carry

    lax.fori_loop(0, cnt, chunk, 0)
    pltpu.sync_copy(s_part, s_out.at[wid])


@functools.partial(
    pl.kernel,
    out_type=jax.ShapeDtypeStruct((NROWS, 128), _f32),
    mesh=_MESH,
    compiler_params=pltpu.CompilerParams(needs_layout_passes=False, use_tc_tiling_on_sc=False),
    scratch_types=[
        pltpu.VMEM((128,), _i32),
        pltpu.VMEM((128,), _f32),
        pltpu.VMEM((N,), _f32),
        pltpu.VMEM((128,), _f32),
    ],
)
def _k4(dst_hbm, w_hbm, r_hbm,
        a_out,
        dst_iv, w_iv, r_v, abuf):
    wid = _wid()
    pltpu.sync_copy(r_hbm, r_v)
    lo, cnt = _chunk_range(wid)

    def chunk(i, carry):
        row = lo + i
        pltpu.sync_copy(dst_hbm.at[row], dst_iv)
        pltpu.sync_copy(w_hbm.at[row], w_iv)
        for g in range(8):
            dst16 = dst_iv[pl.ds(g * 16, 16)]
            w16 = w_iv[pl.ds(g * 16, 16)]
            r16 = plsc.load_gather(r_v, [dst16])
            abuf[pl.ds(g * 16, 16)] = w16 * r16
        pltpu.sync_copy(abuf, a_out.at[row])
        return carry

    lax.fori_loop(0, cnt, chunk, 0)


@functools.partial(
    pl.kernel,
    out_type=jax.ShapeDtypeStruct((2, N, 32), _f32),
    mesh=_MESH,
    compiler_params=pltpu.CompilerParams(needs_layout_passes=False, use_tc_tiling_on_sc=False),
    scratch_types=[
        pltpu.VMEM((128,), _i32),
        pltpu.VMEM((128,), _i32),
        pltpu.VMEM((128,), _f32),
        pltpu.VMEM((128, 32), _f32),
        pltpu.VMEM((128, 32), _f32),
        pltpu.VMEM_SHARED((N, 32), _f32),
        pltpu.SemaphoreType.DMA,
    ],
)
def _k2(t0_hbm, t1_hbm, src_hbm, dst_hbm, a_hbm, zero2_hbm,
        out_hbm,
        src_iv, dst_iv, a_iv, rows_v, sbuf, acc, sem):
    cid = lax.axis_index("c")
    tid = lax.axis_index("s")
    pltpu.sync_copy(zero2_hbm.at[pl.ds(tid * NSLICE, NSLICE)],
                    acc.at[pl.ds(tid * NSLICE, NSLICE)])
    plsc.subcore_barrier()
    lo = tid * (NROWS // 16)
    bidx = [jnp.full((16, 1), l, dtype=_i32) for l in range(16)]
    _dn = lax.GatherDimensionNumbers(offset_dims=(), collapsed_slice_dims=(0,),
                                     start_index_map=(0,))

    def _bcast_lane(vec, l):
        return lax.gather(vec, bidx[l], dimension_numbers=_dn,
                          slice_sizes=(1,),
                          mode=lax.GatherScatterMode.PROMISE_IN_BOUNDS)

    def run(tbl, outslice):
        def chunk(i, carry):
            row = lo + i
            pltpu.sync_copy(src_hbm.at[row], src_iv)
            pltpu.sync_copy(dst_hbm.at[row], dst_iv)
            pltpu.sync_copy(a_hbm.at[row], a_iv)
            pltpu.async_copy(tbl.at[src_iv], rows_v, sem).wait()
            for g in range(8):
                alpha = a_iv[pl.ds(g * 16, 16)]
                for l in range(16):
                    j = g * 16 + l
                    ab = _bcast_lane(alpha, l)
                    sbuf[j, pl.ds(0, 16)] = rows_v[j, pl.ds(0, 16)] * ab
                    sbuf[j, pl.ds(16, 16)] = rows_v[j, pl.ds(16, 16)] * ab
            pltpu.sync_copy(sbuf, acc.at[dst_iv], add=True)
            return carry

        lax.fori_loop(0, NROWS // 16, chunk, 0)
        plsc.subcore_barrier()
        pltpu.sync_copy(acc.at[pl.ds(tid * NSLICE, NSLICE)],
                        outslice.at[pl.ds(tid * NSLICE, NSLICE)])

    @pl.when(cid == 0)
    def _():
        run(t0_hbm, out_hbm.at[0])

    @pl.when(cid == 1)
    def _():
        run(t1_hbm, out_hbm.at[1])


def _lrelu(x, s):
    return jnp.where(x > 0, x, s * x)


_RB = N // 16


def _mm_body(x_ref, w_ref, b_ref, o_ref):
    o_ref[...] = jnp.dot(x_ref[...], w_ref[...],
                         preferred_element_type=_f32) + b_ref[...]


def _mm(x, W, b):
    K = x.shape[1]
    return pl.pallas_call(
        _mm_body,
        grid=(16,),
        in_specs=[pl.BlockSpec((_RB, K), lambda i: (i, 0)),
                  pl.BlockSpec((K, 128), lambda i: (0, 0)),
                  pl.BlockSpec((1, 128), lambda i: (0, 0))],
        out_specs=pl.BlockSpec((_RB, 128), lambda i: (i, 0)),
        out_shape=jax.ShapeDtypeStruct((N, 128), _f32),
    )(x, W, b.reshape(1, 128))


def _ew1_body(o0_ref, o1_ref, b_ref, h_ref, ps_ref):
    h = jnp.concatenate([o0_ref[...], o1_ref[...]], axis=1) + b_ref[...]
    h = _lrelu(h, 0.01)
    h_ref[...] = h
    lane = lax.broadcasted_iota(_i32, (1, 1, 128), 2)
    ps_ref[...] = jnp.where(lane == 0, jnp.sum(h),
                            jnp.where(lane == 1, jnp.sum(h * h), 0.0))


def _ew1(o0, o1, bias):
    return pl.pallas_call(
        _ew1_body,
        grid=(16,),
        in_specs=[pl.BlockSpec((_RB, 32), lambda i: (i, 0)),
                  pl.BlockSpec((_RB, 32), lambda i: (i, 0)),
                  pl.BlockSpec((1, HID), lambda i: (0, 0))],
        out_specs=(pl.BlockSpec((_RB, HID), lambda i: (i, 0)),
                   pl.BlockSpec((1, 1, 128), lambda i: (i, 0, 0))),
        out_shape=(jax.ShapeDtypeStruct((N, HID), _f32),
                   jax.ShapeDtypeStruct((16, 1, 128), _f32)),
    )(o0, o1, bias.reshape(1, HID))


def _ew2_body(o0_ref, o1_ref, xp_ref, wr_ref, b_ref, h_ref):
    agg = jnp.concatenate([o0_ref[...], o1_ref[...]], axis=1)
    h = agg + wr_ref[...] * xp_ref[...] + b_ref[...]
    h_ref[...] = jax.nn.relu(h)


def _ew2(o0, o1, xp, wr, bias):
    return pl.pallas_call(
        _ew2_body,
        grid=(16,),
        in_specs=[pl.BlockSpec((_RB, 32), lambda i: (i, 0)),
                  pl.BlockSpec((_RB, 32), lambda i: (i, 0)),
                  pl.BlockSpec((_RB, HID), lambda i: (i, 0)),
                  pl.BlockSpec((_RB, 1), lambda i: (i, 0)),
                  pl.BlockSpec((1, HID), lambda i: (0, 0))],
        out_specs=pl.BlockSpec((_RB, HID), lambda i: (i, 0)),
        out_shape=jax.ShapeDtypeStruct((N, HID), _f32),
    )(o0, o1, xp, wr.reshape(N, 1), bias.reshape(1, HID))


def _mlp_body(f_ref, w1_ref, b1_ref, w2_ref, b2_ref, o_ref):
    z = jnp.dot(f_ref[...], w1_ref[...], preferred_element_type=_f32) + b1_ref[...]
    z = _lrelu(z, 0.01)
    z = jnp.dot(z, w2_ref[...], preferred_element_type=_f32) + b2_ref[...]
    o_ref[...] = jax.nn.sigmoid(z)


def _mlp(feats, W1, b1, W2p, b2):
    return pl.pallas_call(
        _mlp_body,
        out_shape=jax.ShapeDtypeStruct((391, 128), _f32),
    )(feats, W1, b1.reshape(1, 128), W2p, b2.reshape(1, 1))


def kernel(x, edge_index, batch, W_l, b_l, W_r, b_r, att_v2, bias_v2, gamma,
           beta, Wg1, att_src1, att_dst1, bias_g1, Wg2, att_src2, att_dst2,
           bias_g2, W1, b1, W2, b2):
    src2d = edge_index[0].reshape(NROWS, 128)
    dst2d = edge_index[1].reshape(NROWS, 128)
    zero1 = jnp.zeros((N,), _f32)
    zero2 = jnp.zeros((N, 32), _f32)

    Wlr = jnp.concatenate([W_l, W_r], axis=1)
    blr = jnp.concatenate([b_l, b_r])
    xlr = _mm(x, Wlr, blr)
    xl = xlr[:, :HID]
    xr = xlr[:, HID:]
    att_pad = jnp.concatenate([att_v2, jnp.zeros((64,), _f32)])
    w1, s1p = _k1(xl, xr, att_pad, src2d, dst2d, zero1)
    s1 = s1p.sum(axis=0)
    r1 = 1.0 / (s1 + 1e-16)
    a1 = _k4(dst2d, w1, r1)
    o1 = _k2(xl[:, :32], xl[:, 32:64], src2d, dst2d, a1, zero2)
    h, ps = _ew1(o1[0], o1[1], bias_v2)

    cnt = jnp.float32(N * HID)
    mean = jnp.sum(ps[:, 0, 0]) / cnt
    var = jnp.sum(ps[:, 0, 1]) / cnt - mean * mean
    inv = 1.0 / (jnp.sqrt(var) + 1e-5)
    a_vec = inv * gamma
    b_vec = beta - mean * inv * gamma

    for li, (W, a_s, a_d, bias) in enumerate((
            (Wg1, att_src1, att_dst1, bias_g1),
            (Wg2, att_src2, att_dst2, bias_g2))):
        if li == 0:
            W_eff = a_vec[:, None] * W
            b_eff = b_vec @ W
        else:
            W_eff = W
            b_eff = jnp.zeros((HID,), _f32)
        Waug = jnp.concatenate(
            [W_eff, (W_eff @ a_s)[:, None], (W_eff @ a_d)[:, None],
             jnp.zeros((HID, 62), _f32)], axis=1)
        baug = jnp.concatenate(
            [b_eff, (b_eff @ a_s)[None], (b_eff @ a_d)[None],
             jnp.zeros((62,), _f32)])
        paug = _mm(h, Waug, baug)
        xp = paug[:, :HID]
        asv = paug[:, HID]
        adv = paug[:, HID + 1]
        w_e = _k3(asv, adv, src2d, dst2d)
        sp = _ks(dst2d, w_e, zero1)
        wself = jnp.exp(_lrelu(asv + adv, 0.2))
        s = sp.sum(axis=0) + wself
        r = 1.0 / (s + 1e-16)
        a_e = _k4(dst2d, w_e, r)
        o = _k2(xp[:, :32], xp[:, 32:64], src2d, dst2d, a_e, zero2)
        h = _ew2(o[0], o[1], xp, wself * r, bias)

    feats = h.reshape(391, 128 * HID)
    W2p = jnp.pad(W2, ((0, 0), (0, 127)))
    z = _mlp(feats, W1, b1, W2p, b2)
    return z[:, :1]


# speedup vs baseline: 1.1883x; 1.1883x over previous
import functools

import jax
import jax.numpy as jnp
from jax import lax
from jax.experimental import pallas as pl
from jax.experimental.pallas import tpu as pltpu
from jax.experimental.pallas import tpu_sc as plsc

N = 50048
E = 800768
NROWS = E // 128
HID = 64
NSLICE = N // 16

_MESH = plsc.VectorSubcoreMesh(core_axis_name="c", subcore_axis_name="s")

_f32 = jnp.float32
_i32 = jnp.int32


def _wid():
    return lax.axis_index("s") * 2 + lax.axis_index("c")


def _chunk_range(wid):
    lo = wid * 195 + jnp.minimum(wid, 16)
    cnt = jnp.where(wid < 16, 196, 195)
    return lo, cnt


_IOTA16 = lambda: lax.broadcasted_iota(_i32, (16,), 0)


NR64 = E // 64
RPT = NR64 // 32


@functools.partial(
    pl.kernel,
    out_type=(jax.ShapeDtypeStruct((NR64, 64), _f32),
              jax.ShapeDtypeStruct((32, N), _f32)),
    mesh=_MESH,
    compiler_params=pltpu.CompilerParams(needs_layout_passes=False, use_tc_tiling_on_sc=False),
    scratch_types=[
        pltpu.VMEM((2, 64), _i32),
        pltpu.VMEM((2, 64), _i32),
        pltpu.VMEM((2, 64, HID), _f32),
        pltpu.VMEM((2, 64, HID), _f32),
        pltpu.VMEM((128,), _f32),
        pltpu.VMEM((2, 64), _f32),
        pltpu.VMEM((N,), _f32),
        pltpu.SemaphoreType.DMA,
        pltpu.SemaphoreType.DMA,
        pltpu.SemaphoreType.DMA,
        pltpu.SemaphoreType.DMA,
        pltpu.SemaphoreType.DMA,
    ],
)
def _k1(xl_hbm, xr_hbm, att_hbm, src_hbm, dst_hbm, zero1_hbm,
        w_out, s_out,
        siv, div, srows, drows, att_v, wbuf, s_part,
        semg1a, semg1b, semg2a, semg2b, semw):
    semg1 = (semg1a, semg1b)
    semg2 = (semg2a, semg2b)
    wid = _wid()
    pltpu.sync_copy(att_hbm, att_v)
    pltpu.sync_copy(zero1_hbm, s_part)
    lo = wid * RPT
    iota16 = _IOTA16()

    def fetch(row, s):
        pltpu.sync_copy(src_hbm.at[row], siv.at[s])
        pltpu.sync_copy(dst_hbm.at[row], div.at[s])
        pltpu.async_copy(xl_hbm.at[siv.at[s]], srows.at[s], semg1[s])
        pltpu.async_copy(xr_hbm.at[div.at[s]], drows.at[s], semg2[s])

    def compute(row, s, first):
        pltpu.make_async_copy(xl_hbm.at[siv.at[s]], srows.at[s], semg1[s]).wait()
        pltpu.make_async_copy(xr_hbm.at[div.at[s]], drows.at[s], semg2[s]).wait()

        def dbody(d, accs):
            attb = plsc.load_gather(att_v, [iota16 * 0 + d])
            out = []
            for g in range(4):
                rid = iota16 + g * 16
                cid = iota16 * 0 + d
                s16 = plsc.load_gather(srows.at[s], [rid, cid])
                d16 = plsc.load_gather(drows.at[s], [rid, cid])
                za = s16 + d16
                lr = 0.6 * za + 0.4 * jnp.abs(za)
                out.append(accs[g] + lr * attb)
            return tuple(out)

        accs = lax.fori_loop(0, HID, dbody,
                             tuple(jnp.zeros((16,), _f32) for _ in range(4)))
        @pl.when(jnp.logical_not(first))
        def _():
            pltpu.make_async_copy(wbuf.at[s], w_out.at[row], semw).wait()
        for g in range(4):
            w16 = jnp.exp(accs[g])
            wbuf[s, pl.ds(g * 16, 16)] = w16
            dst16 = div[s, pl.ds(g * 16, 16)]
            plsc.addupdate_scatter(s_part, [dst16], w16)
        pltpu.async_copy(wbuf.at[s], w_out.at[row], semw)

    fetch(lo, 0)

    def pair(p, carry):
        c0 = 2 * p
        fetch(lo + c0 + 1, 1)
        compute(lo + c0, 0, c0 == 0)
        fetch(lo + c0 + 2, 0)
        compute(lo + c0 + 1, 1, False)
        return carry

    lax.fori_loop(0, (RPT - 1) // 2, pair, 0)
    compute(lo + RPT - 1, 0, False)
    pltpu.make_async_copy(wbuf.at[0], w_out.at[lo], semw).wait()
    pltpu.sync_copy(s_part, s_out.at[wid])


@functools.partial(
    pl.kernel,
    out_type=jax.ShapeDtypeStruct((NROWS, 128), _f32),
    mesh=_MESH,
    compiler_params=pltpu.CompilerParams(needs_layout_passes=False, use_tc_tiling_on_sc=False),
    scratch_types=[
        pltpu.VMEM((128,), _i32),
        pltpu.VMEM((128,), _i32),
        pltpu.VMEM((N,), _f32),
        pltpu.VMEM((N,), _f32),
        pltpu.VMEM((128,), _f32),
    ],
)
def _k3(asrc_hbm, adst_hbm, src_hbm, dst_hbm,
        w_out,
        src_iv, dst_iv, asv, adv, wbuf):
    wid = _wid()
    pltpu.sync_copy(asrc_hbm, asv)
    pltpu.sync_copy(adst_hbm, adv)
    lo, cnt = _chunk_range(wid)

    def chunk(i, carry):
        row = lo + i
        pltpu.sync_copy(src_hbm.at[row], src_iv)
        pltpu.sync_copy(dst_hbm.at[row], dst_iv)
        for g in range(8):
            src16 = src_iv[pl.ds(g * 16, 16)]
            dst16 = dst_iv[pl.ds(g * 16, 16)]
            a16 = plsc.load_gather(asv, [src16])
            b16 = plsc.load_gather(adv, [dst16])
            z = a16 + b16
            lr = 0.6 * z + 0.4 * jnp.abs(z)
            wbuf[pl.ds(g * 16, 16)] = jnp.exp(lr)
        pltpu.sync_copy(wbuf, w_out.at[row])
        return carry

    lax.fori_loop(0, cnt, chunk, 0)


@functools.partial(
    pl.kernel,
    out_type=jax.ShapeDtypeStruct((32, N), _f32),
    mesh=_MESH,
    compiler_params=pltpu.CompilerParams(needs_layout_passes=False, use_tc_tiling_on_sc=False),
    scratch_types=[
        pltpu.VMEM((128,), _i32),
        pltpu.VMEM((128,), _f32),
        pltpu.VMEM((N,), _f32),
    ],
)
def _ks(dst_hbm, w_hbm, zero1_hbm,
        s_out,
        dst_iv, w_iv, s_part):
    wid = _wid()
    pltpu.sync_copy(zero1_hbm, s_part)
    lo, cnt = _chunk_range(wid)

    def chunk(i, carry):
        row = lo + i
        pltpu.sync_copy(dst_hbm.at[row], dst_iv)
        pltpu.sync_copy(w_hbm.at[row], w_iv)
        for g in range(8):
            dst16 = dst_iv[pl.ds(g * 16, 16)]
            w16 = w_iv[pl.ds(g * 16, 16)]
            plsc.addupdate_scatter(s_part, [dst16], w16)
        return carry

    lax.fori_loop(0, cnt, chunk, 0)
    pltpu.sync_copy(s_part, s_out.at[wid])


@functools.partial(
    pl.kernel,
    out_type=jax.ShapeDtypeStruct((NROWS, 128), _f32),
    mesh=_MESH,
    compiler_params=pltpu.CompilerParams(needs_layout_passes=False, use_tc_tiling_on_sc=False),
    scratch_types=[
        pltpu.VMEM((128,), _i32),
        pltpu.VMEM((128,), _f32),
        pltpu.VMEM((N,), _f32),
        pltpu.VMEM((128,), _f32),
    ],
)
def _k4(dst_hbm, w_hbm, r_hbm,
        a_out,
        dst_iv, w_iv, r_v, abuf):
    wid = _wid()
    pltpu.sync_copy(r_hbm, r_v)
    lo, cnt = _chunk_range(wid)

    def chunk(i, carry):
        row = lo + i
        pltpu.sync_copy(dst_hbm.at[row], dst_iv)
        pltpu.sync_copy(w_hbm.at[row], w_iv)
        for g in range(8):
            dst16 = dst_iv[pl.ds(g * 16, 16)]
            w16 = w_iv[pl.ds(g * 16, 16)]
            r16 = plsc.load_gather(r_v, [dst16])
            abuf[pl.ds(g * 16, 16)] = w16 * r16
        pltpu.sync_copy(abuf, a_out.at[row])
        return carry

    lax.fori_loop(0, cnt, chunk, 0)


@functools.partial(
    pl.kernel,
    out_type=jax.ShapeDtypeStruct((2, N, 32), _f32),
    mesh=_MESH,
    compiler_params=pltpu.CompilerParams(needs_layout_passes=False, use_tc_tiling_on_sc=False),
    scratch_types=[
        pltpu.VMEM((2, 128), _i32),
        pltpu.VMEM((2, 128), _i32),
        pltpu.VMEM((2, 128), _f32),
        pltpu.VMEM((2, 128, 32), _f32),
        pltpu.VMEM((2, 128, 32), _f32),
        pltpu.VMEM((2, 128), _i32),
        pltpu.VMEM_SHARED((N, 32), _f32),
        pltpu.SemaphoreType.DMA,
        pltpu.SemaphoreType.DMA,
        pltpu.SemaphoreType.DMA,
    ],
)
def _k2(t0_hbm, t1_hbm, src_hbm, dst_hbm, a_hbm, zero2_hbm,
        out_hbm,
        siv, div, aiv, rows_v, sbuf, dsciv, acc, semg0, semg1, sems):
    cid = lax.axis_index("c")
    tid = lax.axis_index("s")
    semg = (semg0, semg1)
    pltpu.sync_copy(zero2_hbm.at[pl.ds(tid * NSLICE, NSLICE)],
                    acc.at[pl.ds(tid * NSLICE, NSLICE)])
    plsc.subcore_barrier()
    lo = tid * (NROWS // 16)
    bidx = [jnp.full((16, 1), l, dtype=_i32) for l in range(16)]
    _dn = lax.GatherDimensionNumbers(offset_dims=(), collapsed_slice_dims=(0,),
                                     start_index_map=(0,))

    def _bcast_lane(vec, l):
        return lax.gather(vec, bidx[l], dimension_numbers=_dn,
                          slice_sizes=(1,),
                          mode=lax.GatherScatterMode.PROMISE_IN_BOUNDS)

    def run(tbl, outslice):
        def fetch(row, s):
            pltpu.sync_copy(src_hbm.at[row], siv.at[s])
            pltpu.sync_copy(dst_hbm.at[row], div.at[s])
            pltpu.sync_copy(a_hbm.at[row], aiv.at[s])
            pltpu.async_copy(tbl.at[siv.at[s]], rows_v.at[s], semg[s])

        def compute(row, s, first):
            @pl.when(jnp.logical_not(first))
            def _():
                pltpu.make_async_copy(sbuf.at[s], acc.at[dsciv.at[s]],
                                      sems).wait()
            pltpu.make_async_copy(tbl.at[siv.at[s]], rows_v.at[s],
                                  semg[s]).wait()
            for g in range(8):
                alpha = aiv[s, pl.ds(g * 16, 16)]
                for l in range(16):
                    j = g * 16 + l
                    ab = _bcast_lane(alpha, l)
                    sbuf[s, j, pl.ds(0, 16)] = rows_v[s, j, pl.ds(0, 16)] * ab
                    sbuf[s, j, pl.ds(16, 16)] = rows_v[s, j, pl.ds(16, 16)] * ab
                dsciv[s, pl.ds(g * 16, 16)] = div[s, pl.ds(g * 16, 16)]
            pltpu.async_copy(sbuf.at[s], acc.at[dsciv.at[s]], sems, add=True)

        fetch(lo, 0)

        def pair(p, carry):
            c0 = 2 * p
            fetch(lo + c0 + 1, 1)
            compute(lo + c0, 0, c0 == 0)
            fetch(lo + c0 + 2, 0)
            compute(lo + c0 + 1, 1, False)
            return carry

        lax.fori_loop(0, (NROWS // 16 - 1) // 2, pair, 0)
        compute(lo + NROWS // 16 - 1, 0, False)
        pltpu.make_async_copy(sbuf.at[0], acc.at[dsciv.at[0]], sems).wait()
        plsc.subcore_barrier()
        pltpu.sync_copy(acc.at[pl.ds(tid * NSLICE, NSLICE)],
                        outslice.at[pl.ds(tid * NSLICE, NSLICE)])

    @pl.when(cid == 0)
    def _():
        run(t0_hbm, out_hbm.at[0])

    @pl.when(cid == 1)
    def _():
        run(t1_hbm, out_hbm.at[1])


def _lrelu(x, s):
    return jnp.where(x > 0, x, s * x)


_RB = N // 16


def _mm_body(x_ref, w_ref, b_ref, o_ref):
    o_ref[...] = jnp.dot(x_ref[...], w_ref[...],
                         preferred_element_type=_f32) + b_ref[...]


def _mm(x, W, b):
    K = x.shape[1]
    return pl.pallas_call(
        _mm_body,
        grid=(16,),
        in_specs=[pl.BlockSpec((_RB, K), lambda i: (i, 0)),
                  pl.BlockSpec((K, 128), lambda i: (0, 0)),
                  pl.BlockSpec((1, 128), lambda i: (0, 0))],
        out_specs=pl.BlockSpec((_RB, 128), lambda i: (i, 0)),
        out_shape=jax.ShapeDtypeStruct((N, 128), _f32),
    )(x, W, b.reshape(1, 128))


def _ew1_body(o0_ref, o1_ref, b_ref, h_ref, ps_ref):
    h = jnp.concatenate([o0_ref[...], o1_ref[...]], axis=1) + b_ref[...]
    h = _lrelu(h, 0.01)
    h_ref[...] = h
    lane = lax.broadcasted_iota(_i32, (1, 1, 128), 2)
    ps_ref[...] = jnp.where(lane == 0, jnp.sum(h),
                            jnp.where(lane == 1, jnp.sum(h * h), 0.0))


def _ew1(o0, o1, bias):
    return pl.pallas_call(
        _ew1_body,
        grid=(16,),
        in_specs=[pl.BlockSpec((_RB, 32), lambda i: (i, 0)),
                  pl.BlockSpec((_RB, 32), lambda i: (i, 0)),
                  pl.BlockSpec((1, HID), lambda i: (0, 0))],
        out_specs=(pl.BlockSpec((_RB, HID), lambda i: (i, 0)),
                   pl.BlockSpec((1, 1, 128), lambda i: (i, 0, 0))),
        out_shape=(jax.ShapeDtypeStruct((N, HID), _f32),
                   jax.ShapeDtypeStruct((16, 1, 128), _f32)),
    )(o0, o1, bias.reshape(1, HID))


def _ew2_body(o0_ref, o1_ref, xp_ref, wr_ref, b_ref, h_ref):
    agg = jnp.concatenate([o0_ref[...], o1_ref[...]], axis=1)
    h = agg + wr_ref[...] * xp_ref[...] + b_ref[...]
    h_ref[...] = jax.nn.relu(h)


def _ew2(o0, o1, xp, wr, bias):
    return pl.pallas_call(
        _ew2_body,
        grid=(16,),
        in_specs=[pl.BlockSpec((_RB, 32), lambda i: (i, 0)),
                  pl.BlockSpec((_RB, 32), lambda i: (i, 0)),
                  pl.BlockSpec((_RB, HID), lambda i: (i, 0)),
                  pl.BlockSpec((_RB, 1), lambda i: (i, 0)),
                  pl.BlockSpec((1, HID), lambda i: (0, 0))],
        out_specs=pl.BlockSpec((_RB, HID), lambda i: (i, 0)),
        out_shape=jax.ShapeDtypeStruct((N, HID), _f32),
    )(o0, o1, xp, wr.reshape(N, 1), bias.reshape(1, HID))


def _mlp_body(f_ref, w1_ref, b1_ref, w2_ref, b2_ref, o_ref):
    z = jnp.dot(f_ref[...], w1_ref[...], preferred_element_type=_f32) + b1_ref[...]
    z = _lrelu(z, 0.01)
    z = jnp.dot(z, w2_ref[...], preferred_element_type=_f32) + b2_ref[...]
    o_ref[...] = jax.nn.sigmoid(z)


def _mlp(feats, W1, b1, W2p, b2):
    return pl.pallas_call(
        _mlp_body,
        out_shape=jax.ShapeDtypeStruct((391, 128), _f32),
    )(feats, W1, b1.reshape(1, 128), W2p, b2.reshape(1, 1))


def kernel(x, edge_index, batch, W_l, b_l, W_r, b_r, att_v2, bias_v2, gamma,
           beta, Wg1, att_src1, att_dst1, bias_g1, Wg2, att_src2, att_dst2,
           bias_g2, W1, b1, W2, b2):
    src2d = edge_index[0].reshape(NROWS, 128)
    dst2d = edge_index[1].reshape(NROWS, 128)
    zero1 = jnp.zeros((N,), _f32)
    zero2 = jnp.zeros((N, 32), _f32)

    Wlr = jnp.concatenate([W_l, W_r], axis=1)
    blr = jnp.concatenate([b_l, b_r])
    xlr = _mm(x, Wlr, blr)
    xl = xlr[:, :HID]
    xr = xlr[:, HID:]
    att_pad = jnp.concatenate([att_v2, jnp.zeros((64,), _f32)])
    src2d64 = edge_index[0].reshape(NR64, 64)
    dst2d64 = edge_index[1].reshape(NR64, 64)
    w1, s1p = _k1(xl, xr, att_pad, src2d64, dst2d64, zero1)
    w1 = w1.reshape(NROWS, 128)
    s1 = s1p.sum(axis=0)
    r1 = 1.0 / (s1 + 1e-16)
    a1 = _k4(dst2d, w1, r1)
    o1 = _k2(xl[:, :32], xl[:, 32:64], src2d, dst2d, a1, zero2)
    h, ps = _ew1(o1[0], o1[1], bias_v2)

    cnt = jnp.float32(N * HID)
    mean = jnp.sum(ps[:, 0, 0]) / cnt
    var = jnp.sum(ps[:, 0, 1]) / cnt - mean * mean
    inv = 1.0 / (jnp.sqrt(var) + 1e-5)
    a_vec = inv * gamma
    b_vec = beta - mean * inv * gamma

    for li, (W, a_s, a_d, bias) in enumerate((
            (Wg1, att_src1, att_dst1, bias_g1),
            (Wg2, att_src2, att_dst2, bias_g2))):
        if li == 0:
            W_eff = a_vec[:, None] * W
            b_eff = b_vec @ W
        else:
            W_eff = W
            b_eff = jnp.zeros((HID,), _f32)
        Waug = jnp.concatenate(
            [W_eff, (W_eff @ a_s)[:, None], (W_eff @ a_d)[:, None],
             jnp.zeros((HID, 62), _f32)], axis=1)
        baug = jnp.concatenate(
            [b_eff, (b_eff @ a_s)[None], (b_eff @ a_d)[None],
             jnp.zeros((62,), _f32)])
        paug = _mm(h, Waug, baug)
        xp = paug[:, :HID]
        asv = paug[:, HID]
        adv = paug[:, HID + 1]
        w_e = _k3(asv, adv, src2d, dst2d)
        sp = _ks(dst2d, w_e, zero1)
        wself = jnp.exp(_lrelu(asv + adv, 0.2))
        s = sp.sum(axis=0) + wself
        r = 1.0 / (s + 1e-16)
        a_e = _k4(dst2d, w_e, r)
        o = _k2(xp[:, :32], xp[:, 32:64], src2d, dst2d, a_e, zero2)
        h = _ew2(o[0], o[1], xp, wself * r, bias)

    feats = h.reshape(391, 128 * HID)
    W2p = jnp.pad(W2, ((0, 0), (0, 127)))
    z = _mlp(feats, W1, b1, W2p, b2)
    return z[:, :1]


# speedup vs baseline: 1.3986x; 1.1770x over previous
import functools

import jax
import jax.numpy as jnp
from jax import lax
from jax.experimental import pallas as pl
from jax.experimental.pallas import tpu as pltpu
from jax.experimental.pallas import tpu_sc as plsc

N = 50048
E = 800768
NROWS = E // 128
HID = 64
NSLICE = N // 16

_MESH = plsc.VectorSubcoreMesh(core_axis_name="c", subcore_axis_name="s")

_f32 = jnp.float32
_i32 = jnp.int32


def _wid():
    return lax.axis_index("s") * 2 + lax.axis_index("c")


def _chunk_range(wid):
    lo = wid * 195 + jnp.minimum(wid, 16)
    cnt = jnp.where(wid < 16, 196, 195)
    return lo, cnt


_IOTA16 = lambda: lax.broadcasted_iota(_i32, (16,), 0)


NR64 = E // 64
RPT = NR64 // 32


@functools.partial(
    pl.kernel,
    out_type=(jax.ShapeDtypeStruct((NR64, 64), _f32),
              jax.ShapeDtypeStruct((32, N), _f32)),
    mesh=_MESH,
    compiler_params=pltpu.CompilerParams(needs_layout_passes=False, use_tc_tiling_on_sc=False),
    scratch_types=[
        pltpu.VMEM((2, 64), _i32),
        pltpu.VMEM((2, 64), _i32),
        pltpu.VMEM((2, 64, HID), _f32),
        pltpu.VMEM((2, 64, HID), _f32),
        pltpu.VMEM((128,), _f32),
        pltpu.VMEM((2, 64), _f32),
        pltpu.VMEM((N,), _f32),
        pltpu.SemaphoreType.DMA,
        pltpu.SemaphoreType.DMA,
        pltpu.SemaphoreType.DMA,
        pltpu.SemaphoreType.DMA,
        pltpu.SemaphoreType.DMA,
    ],
)
def _k1(xl_hbm, xr_hbm, att_hbm, src_hbm, dst_hbm, zero1_hbm,
        w_out, s_out,
        siv, div, srows, drows, att_v, wbuf, s_part,
        semg1a, semg1b, semg2a, semg2b, semw):
    semg1 = (semg1a, semg1b)
    semg2 = (semg2a, semg2b)
    wid = _wid()
    pltpu.sync_copy(att_hbm, att_v)
    pltpu.sync_copy(zero1_hbm, s_part)
    lo = wid * RPT
    iota16 = _IOTA16()

    def fetch(row, s):
        pltpu.sync_copy(src_hbm.at[row], siv.at[s])
        pltpu.sync_copy(dst_hbm.at[row], div.at[s])
        pltpu.async_copy(xl_hbm.at[siv.at[s]], srows.at[s], semg1[s])
        pltpu.async_copy(xr_hbm.at[div.at[s]], drows.at[s], semg2[s])

    def compute(row, s, first):
        pltpu.make_async_copy(xl_hbm.at[siv.at[s]], srows.at[s], semg1[s]).wait()
        pltpu.make_async_copy(xr_hbm.at[div.at[s]], drows.at[s], semg2[s]).wait()

        def dbody(d, accs):
            attb = plsc.load_gather(att_v, [iota16 * 0 + d])
            out = []
            for g in range(4):
                rid = iota16 + g * 16
                cid = iota16 * 0 + d
                s16 = plsc.load_gather(srows.at[s], [rid, cid])
                d16 = plsc.load_gather(drows.at[s], [rid, cid])
                za = s16 + d16
                lr = 0.6 * za + 0.4 * jnp.abs(za)
                out.append(accs[g] + lr * attb)
            return tuple(out)

        accs = lax.fori_loop(0, HID, dbody,
                             tuple(jnp.zeros((16,), _f32) for _ in range(4)))
        @pl.when(jnp.logical_not(first))
        def _():
            pltpu.make_async_copy(wbuf.at[s], w_out.at[row], semw).wait()
        for g in range(4):
            w16 = jnp.exp(accs[g])
            wbuf[s, pl.ds(g * 16, 16)] = w16
            dst16 = div[s, pl.ds(g * 16, 16)]
            plsc.addupdate_scatter(s_part, [dst16], w16)
        pltpu.async_copy(wbuf.at[s], w_out.at[row], semw)

    fetch(lo, 0)

    def pair(p, carry):
        c0 = 2 * p
        fetch(lo + c0 + 1, 1)
        compute(lo + c0, 0, c0 == 0)
        fetch(lo + c0 + 2, 0)
        compute(lo + c0 + 1, 1, False)
        return carry

    lax.fori_loop(0, (RPT - 1) // 2, pair, 0)
    compute(lo + RPT - 1, 0, False)
    pltpu.make_async_copy(wbuf.at[0], w_out.at[lo], semw).wait()
    pltpu.sync_copy(s_part, s_out.at[wid])


@functools.partial(
    pl.kernel,
    out_type=jax.ShapeDtypeStruct((NR64, 64), _f32),
    mesh=_MESH,
    compiler_params=pltpu.CompilerParams(needs_layout_passes=False, use_tc_tiling_on_sc=False),
    scratch_types=[
        pltpu.VMEM((8, 64), _i32),
        pltpu.VMEM((8, 64), _i32),
        pltpu.VMEM((N,), _f32),
        pltpu.VMEM((N,), _f32),
        pltpu.VMEM((8, 64), _f32),
    ],
)
def _k3(asrc_hbm, adst_hbm, src_hbm, dst_hbm,
        w_out,
        src_iv, dst_iv, asv, adv, wbuf):
    wid = _wid()
    pltpu.sync_copy(asrc_hbm, asv)
    pltpu.sync_copy(adst_hbm, adv)
    lo = wid * RPT

    def batch(row, nrow):
        pltpu.sync_copy(src_hbm.at[pl.ds(row, nrow)], src_iv.at[pl.ds(0, nrow)])
        pltpu.sync_copy(dst_hbm.at[pl.ds(row, nrow)], dst_iv.at[pl.ds(0, nrow)])
        for q in range(nrow):
            for g in range(4):
                src16 = src_iv[q, pl.ds(g * 16, 16)]
                dst16 = dst_iv[q, pl.ds(g * 16, 16)]
                a16 = plsc.load_gather(asv, [src16])
                b16 = plsc.load_gather(adv, [dst16])
                z = a16 + b16
                lr = 0.6 * z + 0.4 * jnp.abs(z)
                wbuf[q, pl.ds(g * 16, 16)] = jnp.exp(lr)
        pltpu.sync_copy(wbuf.at[pl.ds(0, nrow)], w_out.at[pl.ds(row, nrow)])

    def chunk(i, carry):
        batch(lo + i * 8, 8)
        return carry

    lax.fori_loop(0, RPT // 8, chunk, 0)
    batch(lo + (RPT // 8) * 8, RPT % 8)


@functools.partial(
    pl.kernel,
    out_type=jax.ShapeDtypeStruct((32, N), _f32),
    mesh=_MESH,
    compiler_params=pltpu.CompilerParams(needs_layout_passes=False, use_tc_tiling_on_sc=False),
    scratch_types=[
        pltpu.VMEM((8, 64), _i32),
        pltpu.VMEM((8, 64), _f32),
        pltpu.VMEM((N,), _f32),
    ],
)
def _ks(dst_hbm, w_hbm, zero1_hbm,
        s_out,
        dst_iv, w_iv, s_part):
    wid = _wid()
    pltpu.sync_copy(zero1_hbm, s_part)
    lo = wid * RPT

    def batch(row, nrow):
        pltpu.sync_copy(dst_hbm.at[pl.ds(row, nrow)], dst_iv.at[pl.ds(0, nrow)])
        pltpu.sync_copy(w_hbm.at[pl.ds(row, nrow)], w_iv.at[pl.ds(0, nrow)])
        for q in range(nrow):
            for g in range(4):
                dst16 = dst_iv[q, pl.ds(g * 16, 16)]
                w16 = w_iv[q, pl.ds(g * 16, 16)]
                plsc.addupdate_scatter(s_part, [dst16], w16)

    def chunk(i, carry):
        batch(lo + i * 8, 8)
        return carry

    lax.fori_loop(0, RPT // 8, chunk, 0)
    batch(lo + (RPT // 8) * 8, RPT % 8)
    pltpu.sync_copy(s_part, s_out.at[wid])


@functools.partial(
    pl.kernel,
    out_type=jax.ShapeDtypeStruct((NR64, 64), _f32),
    mesh=_MESH,
    compiler_params=pltpu.CompilerParams(needs_layout_passes=False, use_tc_tiling_on_sc=False),
    scratch_types=[
        pltpu.VMEM((8, 64), _i32),
        pltpu.VMEM((8, 64), _f32),
        pltpu.VMEM((N,), _f32),
        pltpu.VMEM((8, 64), _f32),
    ],
)
def _k4(dst_hbm, w_hbm, r_hbm,
        a_out,
        dst_iv, w_iv, r_v, abuf):
    wid = _wid()
    pltpu.sync_copy(r_hbm, r_v)
    lo = wid * RPT

    def batch(row, nrow):
        pltpu.sync_copy(dst_hbm.at[pl.ds(row, nrow)], dst_iv.at[pl.ds(0, nrow)])
        pltpu.sync_copy(w_hbm.at[pl.ds(row, nrow)], w_iv.at[pl.ds(0, nrow)])
        for q in range(nrow):
            for g in range(4):
                dst16 = dst_iv[q, pl.ds(g * 16, 16)]
                w16 = w_iv[q, pl.ds(g * 16, 16)]
                r16 = plsc.load_gather(r_v, [dst16])
                abuf[q, pl.ds(g * 16, 16)] = w16 * r16
        pltpu.sync_copy(abuf.at[pl.ds(0, nrow)], a_out.at[pl.ds(row, nrow)])

    def chunk(i, carry):
        batch(lo + i * 8, 8)
        return carry

    lax.fori_loop(0, RPT // 8, chunk, 0)
    batch(lo + (RPT // 8) * 8, RPT % 8)


@functools.partial(
    pl.kernel,
    out_type=jax.ShapeDtypeStruct((2, N, 32), _f32),
    mesh=_MESH,
    compiler_params=pltpu.CompilerParams(needs_layout_passes=False, use_tc_tiling_on_sc=False),
    scratch_types=[
        pltpu.VMEM((2, 128), _i32),
        pltpu.VMEM((2, 128), _i32),
        pltpu.VMEM((2, 128), _f32),
        pltpu.VMEM((2, 128, 32), _f32),
        pltpu.VMEM((2, 128, 32), _f32),
        pltpu.VMEM((2, 128), _i32),
        pltpu.VMEM_SHARED((N, 32), _f32),
        pltpu.SemaphoreType.DMA,
        pltpu.SemaphoreType.DMA,
        pltpu.SemaphoreType.DMA,
    ],
)
def _k2(t0_hbm, t1_hbm, src_hbm, dst_hbm, a_hbm, zero2_hbm,
        out_hbm,
        siv, div, aiv, rows_v, sbuf, dsciv, acc, semg0, semg1, sems):
    cid = lax.axis_index("c")
    tid = lax.axis_index("s")
    semg = (semg0, semg1)
    pltpu.sync_copy(zero2_hbm.at[pl.ds(tid * NSLICE, NSLICE)],
                    acc.at[pl.ds(tid * NSLICE, NSLICE)])
    plsc.subcore_barrier()
    lo = tid * (NROWS // 16)
    bidx = [jnp.full((16, 1), l, dtype=_i32) for l in range(16)]
    _dn = lax.GatherDimensionNumbers(offset_dims=(), collapsed_slice_dims=(0,),
                                     start_index_map=(0,))

    def _bcast_lane(vec, l):
        return lax.gather(vec, bidx[l], dimension_numbers=_dn,
                          slice_sizes=(1,),
                          mode=lax.GatherScatterMode.PROMISE_IN_BOUNDS)

    def run(tbl, outslice):
        def fetch(row, s):
            pltpu.sync_copy(src_hbm.at[row], siv.at[s])
            pltpu.sync_copy(dst_hbm.at[row], div.at[s])
            pltpu.sync_copy(a_hbm.at[row], aiv.at[s])
            pltpu.async_copy(tbl.at[siv.at[s]], rows_v.at[s], semg[s])

        def compute(row, s, first):
            @pl.when(jnp.logical_not(first))
            def _():
                pltpu.make_async_copy(sbuf.at[s], acc.at[dsciv.at[s]],
                                      sems).wait()
            pltpu.make_async_copy(tbl.at[siv.at[s]], rows_v.at[s],
                                  semg[s]).wait()
            for g in range(8):
                alpha = aiv[s, pl.ds(g * 16, 16)]
                for l in range(16):
                    j = g * 16 + l
                    ab = _bcast_lane(alpha, l)
                    sbuf[s, j, pl.ds(0, 16)] = rows_v[s, j, pl.ds(0, 16)] * ab
                    sbuf[s, j, pl.ds(16, 16)] = rows_v[s, j, pl.ds(16, 16)] * ab
                dsciv[s, pl.ds(g * 16, 16)] = div[s, pl.ds(g * 16, 16)]
            pltpu.async_copy(sbuf.at[s], acc.at[dsciv.at[s]], sems, add=True)

        fetch(lo, 0)

        def pair(p, carry):
            c0 = 2 * p
            fetch(lo + c0 + 1, 1)
            compute(lo + c0, 0, c0 == 0)
            fetch(lo + c0 + 2, 0)
            compute(lo + c0 + 1, 1, False)
            return carry

        lax.fori_loop(0, (NROWS // 16 - 1) // 2, pair, 0)
        compute(lo + NROWS // 16 - 1, 0, False)
        pltpu.make_async_copy(sbuf.at[0], acc.at[dsciv.at[0]], sems).wait()
        plsc.subcore_barrier()
        pltpu.sync_copy(acc.at[pl.ds(tid * NSLICE, NSLICE)],
                        outslice.at[pl.ds(tid * NSLICE, NSLICE)])

    @pl.when(cid == 0)
    def _():
        run(t0_hbm, out_hbm.at[0])

    @pl.when(cid == 1)
    def _():
        run(t1_hbm, out_hbm.at[1])


def _lrelu(x, s):
    return jnp.where(x > 0, x, s * x)


_RB = N // 16


def _mm_body(x_ref, w_ref, b_ref, o_ref):
    o_ref[...] = jnp.dot(x_ref[...], w_ref[...],
                         preferred_element_type=_f32) + b_ref[...]


def _mm(x, W, b):
    K = x.shape[1]
    return pl.pallas_call(
        _mm_body,
        grid=(16,),
        in_specs=[pl.BlockSpec((_RB, K), lambda i: (i, 0)),
                  pl.BlockSpec((K, 128), lambda i: (0, 0)),
                  pl.BlockSpec((1, 128), lambda i: (0, 0))],
        out_specs=pl.BlockSpec((_RB, 128), lambda i: (i, 0)),
        out_shape=jax.ShapeDtypeStruct((N, 128), _f32),
    )(x, W, b.reshape(1, 128))


def _ew1_body(o0_ref, o1_ref, b_ref, h_ref, ps_ref):
    h = jnp.concatenate([o0_ref[...], o1_ref[...]], axis=1) + b_ref[...]
    h = _lrelu(h, 0.01)
    h_ref[...] = h
    lane = lax.broadcasted_iota(_i32, (1, 1, 128), 2)
    ps_ref[...] = jnp.where(lane == 0, jnp.sum(h),
                            jnp.where(lane == 1, jnp.sum(h * h), 0.0))


def _ew1(o0, o1, bias):
    return pl.pallas_call(
        _ew1_body,
        grid=(16,),
        in_specs=[pl.BlockSpec((_RB, 32), lambda i: (i, 0)),
                  pl.BlockSpec((_RB, 32), lambda i: (i, 0)),
                  pl.BlockSpec((1, HID), lambda i: (0, 0))],
        out_specs=(pl.BlockSpec((_RB, HID), lambda i: (i, 0)),
                   pl.BlockSpec((1, 1, 128), lambda i: (i, 0, 0))),
        out_shape=(jax.ShapeDtypeStruct((N, HID), _f32),
                   jax.ShapeDtypeStruct((16, 1, 128), _f32)),
    )(o0, o1, bias.reshape(1, HID))


def _ew2_body(o0_ref, o1_ref, xp_ref, wr_ref, b_ref, h_ref):
    agg = jnp.concatenate([o0_ref[...], o1_ref[...]], axis=1)
    h = agg + wr_ref[...] * xp_ref[...] + b_ref[...]
    h_ref[...] = jax.nn.relu(h)


def _ew2(o0, o1, xp, wr, bias):
    return pl.pallas_call(
        _ew2_body,
        grid=(16,),
        in_specs=[pl.BlockSpec((_RB, 32), lambda i: (i, 0)),
                  pl.BlockSpec((_RB, 32), lambda i: (i, 0)),
                  pl.BlockSpec((_RB, HID), lambda i: (i, 0)),
                  pl.BlockSpec((_RB, 1), lambda i: (i, 0)),
                  pl.BlockSpec((1, HID), lambda i: (0, 0))],
        out_specs=pl.BlockSpec((_RB, HID), lambda i: (i, 0)),
        out_shape=jax.ShapeDtypeStruct((N, HID), _f32),
    )(o0, o1, xp, wr.reshape(N, 1), bias.reshape(1, HID))


def _mlp_body(f_ref, w1_ref, b1_ref, w2_ref, b2_ref, o_ref):
    z = jnp.dot(f_ref[...], w1_ref[...], preferred_element_type=_f32) + b1_ref[...]
    z = _lrelu(z, 0.01)
    z = jnp.dot(z, w2_ref[...], preferred_element_type=_f32) + b2_ref[...]
    o_ref[...] = jax.nn.sigmoid(z)


def _mlp(feats, W1, b1, W2p, b2):
    return pl.pallas_call(
        _mlp_body,
        out_shape=jax.ShapeDtypeStruct((391, 128), _f32),
    )(feats, W1, b1.reshape(1, 128), W2p, b2.reshape(1, 1))


def kernel(x, edge_index, batch, W_l, b_l, W_r, b_r, att_v2, bias_v2, gamma,
           beta, Wg1, att_src1, att_dst1, bias_g1, Wg2, att_src2, att_dst2,
           bias_g2, W1, b1, W2, b2):
    src2d = edge_index[0].reshape(NROWS, 128)
    dst2d = edge_index[1].reshape(NROWS, 128)
    zero1 = jnp.zeros((N,), _f32)
    zero2 = jnp.zeros((N, 32), _f32)

    Wlr = jnp.concatenate([W_l, W_r], axis=1)
    blr = jnp.concatenate([b_l, b_r])
    xlr = _mm(x, Wlr, blr)
    xl = xlr[:, :HID]
    xr = xlr[:, HID:]
    att_pad = jnp.concatenate([att_v2, jnp.zeros((64,), _f32)])
    src2d64 = edge_index[0].reshape(NR64, 64)
    dst2d64 = edge_index[1].reshape(NR64, 64)
    w1, s1p = _k1(xl, xr, att_pad, src2d64, dst2d64, zero1)
    s1 = s1p.sum(axis=0)
    r1 = 1.0 / (s1 + 1e-16)
    a1 = _k4(dst2d64, w1, r1).reshape(NROWS, 128)
    o1 = _k2(xl[:, :32], xl[:, 32:64], src2d, dst2d, a1, zero2)
    h, ps = _ew1(o1[0], o1[1], bias_v2)

    cnt = jnp.float32(N * HID)
    mean = jnp.sum(ps[:, 0, 0]) / cnt
    var = jnp.sum(ps[:, 0, 1]) / cnt - mean * mean
    inv = 1.0 / (jnp.sqrt(var) + 1e-5)
    a_vec = inv * gamma
    b_vec = beta - mean * inv * gamma

    for li, (W, a_s, a_d, bias) in enumerate((
            (Wg1, att_src1, att_dst1, bias_g1),
            (Wg2, att_src2, att_dst2, bias_g2))):
        if li == 0:
            W_eff = a_vec[:, None] * W
            b_eff = b_vec @ W
        else:
            W_eff = W
            b_eff = jnp.zeros((HID,), _f32)
        Waug = jnp.concatenate(
            [W_eff, (W_eff @ a_s)[:, None], (W_eff @ a_d)[:, None],
             jnp.zeros((HID, 62), _f32)], axis=1)
        baug = jnp.concatenate(
            [b_eff, (b_eff @ a_s)[None], (b_eff @ a_d)[None],
             jnp.zeros((62,), _f32)])
        paug = _mm(h, Waug, baug)
        xp = paug[:, :HID]
        asv = paug[:, HID]
        adv = paug[:, HID + 1]
        w_e = _k3(asv, adv, src2d64, dst2d64)
        sp = _ks(dst2d64, w_e, zero1)
        wself = jnp.exp(_lrelu(asv + adv, 0.2))
        s = sp.sum(axis=0) + wself
        r = 1.0 / (s + 1e-16)
        a_e = _k4(dst2d64, w_e, r).reshape(NROWS, 128)
        o = _k2(xp[:, :32], xp[:, 32:64], src2d, dst2d, a_e, zero2)
        h = _ew2(o[0], o[1], xp, wself * r, bias)

    feats = h.reshape(391, 128 * HID)
    W2p = jnp.pad(W2, ((0, 0), (0, 127)))
    z = _mlp(feats, W1, b1, W2p, b2)
    return z[:, :1]


# speedup vs baseline: 1.4209x; 1.0159x over previous
import functools

import jax
import jax.numpy as jnp
from jax import lax
from jax.experimental import pallas as pl
from jax.experimental.pallas import tpu as pltpu
from jax.experimental.pallas import tpu_sc as plsc

N = 50048
E = 800768
NROWS = E // 128
HID = 64
NSLICE = N // 16

_MESH = plsc.VectorSubcoreMesh(core_axis_name="c", subcore_axis_name="s")

_f32 = jnp.float32
_i32 = jnp.int32


def _wid():
    return lax.axis_index("s") * 2 + lax.axis_index("c")


def _chunk_range(wid):
    lo = wid * 195 + jnp.minimum(wid, 16)
    cnt = jnp.where(wid < 16, 196, 195)
    return lo, cnt


_IOTA16 = lambda: lax.broadcasted_iota(_i32, (16,), 0)


NR64 = E // 64
RPT = NR64 // 32


@functools.partial(
    pl.kernel,
    out_type=(jax.ShapeDtypeStruct((NR64, 64), _f32),
              jax.ShapeDtypeStruct((32, N), _f32)),
    mesh=_MESH,
    compiler_params=pltpu.CompilerParams(needs_layout_passes=False, use_tc_tiling_on_sc=False),
    scratch_types=[
        pltpu.VMEM((2, 64), _i32),
        pltpu.VMEM((2, 64), _i32),
        pltpu.VMEM((2, 64, HID), _f32),
        pltpu.VMEM((2, 64, HID), _f32),
        pltpu.VMEM((128,), _f32),
        pltpu.VMEM((2, 64), _f32),
        pltpu.VMEM((N,), _f32),
        pltpu.SemaphoreType.DMA,
        pltpu.SemaphoreType.DMA,
        pltpu.SemaphoreType.DMA,
        pltpu.SemaphoreType.DMA,
        pltpu.SemaphoreType.DMA,
    ],
)
def _k1(xl_hbm, xr_hbm, att_hbm, src_hbm, dst_hbm, zero1_hbm,
        w_out, s_out,
        siv, div, srows, drows, att_v, wbuf, s_part,
        semg1a, semg1b, semg2a, semg2b, semw):
    semg1 = (semg1a, semg1b)
    semg2 = (semg2a, semg2b)
    wid = _wid()
    pltpu.sync_copy(att_hbm, att_v)
    pltpu.sync_copy(zero1_hbm, s_part)
    lo = wid * RPT
    iota16 = _IOTA16()

    def fetch(row, s):
        pltpu.sync_copy(src_hbm.at[row], siv.at[s])
        pltpu.sync_copy(dst_hbm.at[row], div.at[s])
        pltpu.async_copy(xl_hbm.at[siv.at[s]], srows.at[s], semg1[s])
        pltpu.async_copy(xr_hbm.at[div.at[s]], drows.at[s], semg2[s])

    def compute(row, s, first):
        pltpu.make_async_copy(xl_hbm.at[siv.at[s]], srows.at[s], semg1[s]).wait()
        pltpu.make_async_copy(xr_hbm.at[div.at[s]], drows.at[s], semg2[s]).wait()

        def dbody(d4, accs):
            out = list(accs)
            for dd in range(4):
                d = d4 * 4 + dd
                cid = iota16 * 0 + d
                attb = plsc.load_gather(att_v, [cid])
                for g in range(4):
                    rid = iota16 + g * 16
                    s16 = plsc.load_gather(srows.at[s], [rid, cid])
                    d16 = plsc.load_gather(drows.at[s], [rid, cid])
                    za = s16 + d16
                    lr = 0.6 * za + 0.4 * jnp.abs(za)
                    out[g] = out[g] + lr * attb
            return tuple(out)

        accs = lax.fori_loop(0, HID // 4, dbody,
                             tuple(jnp.zeros((16,), _f32) for _ in range(4)))
        @pl.when(jnp.logical_not(first))
        def _():
            pltpu.make_async_copy(wbuf.at[s], w_out.at[row], semw).wait()
        for g in range(4):
            w16 = jnp.exp(accs[g])
            wbuf[s, pl.ds(g * 16, 16)] = w16
            dst16 = div[s, pl.ds(g * 16, 16)]
            plsc.addupdate_scatter(s_part, [dst16], w16)
        pltpu.async_copy(wbuf.at[s], w_out.at[row], semw)

    fetch(lo, 0)

    def pair(p, carry):
        c0 = 2 * p
        fetch(lo + c0 + 1, 1)
        compute(lo + c0, 0, c0 == 0)
        fetch(lo + c0 + 2, 0)
        compute(lo + c0 + 1, 1, False)
        return carry

    lax.fori_loop(0, (RPT - 1) // 2, pair, 0)
    compute(lo + RPT - 1, 0, False)
    pltpu.make_async_copy(wbuf.at[0], w_out.at[lo], semw).wait()
    pltpu.sync_copy(s_part, s_out.at[wid])


@functools.partial(
    pl.kernel,
    out_type=jax.ShapeDtypeStruct((NR64, 64), _f32),
    mesh=_MESH,
    compiler_params=pltpu.CompilerParams(needs_layout_passes=False, use_tc_tiling_on_sc=False),
    scratch_types=[
        pltpu.VMEM((8, 64), _i32),
        pltpu.VMEM((8, 64), _i32),
        pltpu.VMEM((N,), _f32),
        pltpu.VMEM((N,), _f32),
        pltpu.VMEM((8, 64), _f32),
    ],
)
def _k3(asrc_hbm, adst_hbm, src_hbm, dst_hbm,
        w_out,
        src_iv, dst_iv, asv, adv, wbuf):
    wid = _wid()
    pltpu.sync_copy(asrc_hbm, asv)
    pltpu.sync_copy(adst_hbm, adv)
    lo = wid * RPT

    def batch(row, nrow):
        pltpu.sync_copy(src_hbm.at[pl.ds(row, nrow)], src_iv.at[pl.ds(0, nrow)])
        pltpu.sync_copy(dst_hbm.at[pl.ds(row, nrow)], dst_iv.at[pl.ds(0, nrow)])
        for q in range(nrow):
            for g in range(4):
                src16 = src_iv[q, pl.ds(g * 16, 16)]
                dst16 = dst_iv[q, pl.ds(g * 16, 16)]
                a16 = plsc.load_gather(asv, [src16])
                b16 = plsc.load_gather(adv, [dst16])
                z = a16 + b16
                lr = 0.6 * z + 0.4 * jnp.abs(z)
                wbuf[q, pl.ds(g * 16, 16)] = jnp.exp(lr)
        pltpu.sync_copy(wbuf.at[pl.ds(0, nrow)], w_out.at[pl.ds(row, nrow)])

    def chunk(i, carry):
        batch(lo + i * 8, 8)
        return carry

    lax.fori_loop(0, RPT // 8, chunk, 0)
    batch(lo + (RPT // 8) * 8, RPT % 8)


@functools.partial(
    pl.kernel,
    out_type=jax.ShapeDtypeStruct((32, N), _f32),
    mesh=_MESH,
    compiler_params=pltpu.CompilerParams(needs_layout_passes=False, use_tc_tiling_on_sc=False),
    scratch_types=[
        pltpu.VMEM((8, 64), _i32),
        pltpu.VMEM((8, 64), _f32),
        pltpu.VMEM((N,), _f32),
    ],
)
def _ks(dst_hbm, w_hbm, zero1_hbm,
        s_out,
        dst_iv, w_iv, s_part):
    wid = _wid()
    pltpu.sync_copy(zero1_hbm, s_part)
    lo = wid * RPT

    def batch(row, nrow):
        pltpu.sync_copy(dst_hbm.at[pl.ds(row, nrow)], dst_iv.at[pl.ds(0, nrow)])
        pltpu.sync_copy(w_hbm.at[pl.ds(row, nrow)], w_iv.at[pl.ds(0, nrow)])
        for q in range(nrow):
            for g in range(4):
                dst16 = dst_iv[q, pl.ds(g * 16, 16)]
                w16 = w_iv[q, pl.ds(g * 16, 16)]
                plsc.addupdate_scatter(s_part, [dst16], w16)

    def chunk(i, carry):
        batch(lo + i * 8, 8)
        return carry

    lax.fori_loop(0, RPT // 8, chunk, 0)
    batch(lo + (RPT // 8) * 8, RPT % 8)
    pltpu.sync_copy(s_part, s_out.at[wid])


@functools.partial(
    pl.kernel,
    out_type=jax.ShapeDtypeStruct((NR64, 64), _f32),
    mesh=_MESH,
    compiler_params=pltpu.CompilerParams(needs_layout_passes=False, use_tc_tiling_on_sc=False),
    scratch_types=[
        pltpu.VMEM((8, 64), _i32),
        pltpu.VMEM((8, 64), _f32),
        pltpu.VMEM((N,), _f32),
        pltpu.VMEM((8, 64), _f32),
    ],
)
def _k4(dst_hbm, w_hbm, r_hbm,
        a_out,
        dst_iv, w_iv, r_v, abuf):
    wid = _wid()
    pltpu.sync_copy(r_hbm, r_v)
    lo = wid * RPT

    def batch(row, nrow):
        pltpu.sync_copy(dst_hbm.at[pl.ds(row, nrow)], dst_iv.at[pl.ds(0, nrow)])
        pltpu.sync_copy(w_hbm.at[pl.ds(row, nrow)], w_iv.at[pl.ds(0, nrow)])
        for q in range(nrow):
            for g in range(4):
                dst16 = dst_iv[q, pl.ds(g * 16, 16)]
                w16 = w_iv[q, pl.ds(g * 16, 16)]
                r16 = plsc.load_gather(r_v, [dst16])
                abuf[q, pl.ds(g * 16, 16)] = w16 * r16
        pltpu.sync_copy(abuf.at[pl.ds(0, nrow)], a_out.at[pl.ds(row, nrow)])

    def chunk(i, carry):
        batch(lo + i * 8, 8)
        return carry

    lax.fori_loop(0, RPT // 8, chunk, 0)
    batch(lo + (RPT // 8) * 8, RPT % 8)


@functools.partial(
    pl.kernel,
    out_type=jax.ShapeDtypeStruct((2, N, 32), _f32),
    mesh=_MESH,
    compiler_params=pltpu.CompilerParams(needs_layout_passes=False, use_tc_tiling_on_sc=False),
    scratch_types=[
        pltpu.VMEM((2, 128), _i32),
        pltpu.VMEM((2, 128), _i32),
        pltpu.VMEM((2, 128), _f32),
        pltpu.VMEM((2, 128, 32), _f32),
        pltpu.VMEM((2, 128, 32), _f32),
        pltpu.VMEM((2, 128), _i32),
        pltpu.VMEM_SHARED((N, 32), _f32),
        pltpu.SemaphoreType.DMA,
        pltpu.SemaphoreType.DMA,
        pltpu.SemaphoreType.DMA,
    ],
)
def _k2(t0_hbm, t1_hbm, src_hbm, dst_hbm, a_hbm, zero2_hbm,
        out_hbm,
        siv, div, aiv, rows_v, sbuf, dsciv, acc, semg0, semg1, sems):
    cid = lax.axis_index("c")
    tid = lax.axis_index("s")
    semg = (semg0, semg1)
    pltpu.sync_copy(zero2_hbm.at[pl.ds(tid * NSLICE, NSLICE)],
                    acc.at[pl.ds(tid * NSLICE, NSLICE)])
    plsc.subcore_barrier()
    lo = tid * (NROWS // 16)
    bidx = [jnp.full((16, 1), l, dtype=_i32) for l in range(16)]
    _dn = lax.GatherDimensionNumbers(offset_dims=(), collapsed_slice_dims=(0,),
                                     start_index_map=(0,))

    def _bcast_lane(vec, l):
        return lax.gather(vec, bidx[l], dimension_numbers=_dn,
                          slice_sizes=(1,),
                          mode=lax.GatherScatterMode.PROMISE_IN_BOUNDS)

    def run(tbl, outslice):
        def fetch(row, s):
            pltpu.sync_copy(src_hbm.at[row], siv.at[s])
            pltpu.sync_copy(dst_hbm.at[row], div.at[s])
            pltpu.sync_copy(a_hbm.at[row], aiv.at[s])
            pltpu.async_copy(tbl.at[siv.at[s]], rows_v.at[s], semg[s])

        def compute(row, s, first):
            @pl.when(jnp.logical_not(first))
            def _():
                pltpu.make_async_copy(sbuf.at[s], acc.at[dsciv.at[s]],
                                      sems).wait()
            pltpu.make_async_copy(tbl.at[siv.at[s]], rows_v.at[s],
                                  semg[s]).wait()
            for g in range(8):
                alpha = aiv[s, pl.ds(g * 16, 16)]
                for l in range(16):
                    j = g * 16 + l
                    ab = _bcast_lane(alpha, l)
                    sbuf[s, j, pl.ds(0, 16)] = rows_v[s, j, pl.ds(0, 16)] * ab
                    sbuf[s, j, pl.ds(16, 16)] = rows_v[s, j, pl.ds(16, 16)] * ab
                dsciv[s, pl.ds(g * 16, 16)] = div[s, pl.ds(g * 16, 16)]
            pltpu.async_copy(sbuf.at[s], acc.at[dsciv.at[s]], sems, add=True)

        fetch(lo, 0)

        def pair(p, carry):
            c0 = 2 * p
            fetch(lo + c0 + 1, 1)
            compute(lo + c0, 0, c0 == 0)
            fetch(lo + c0 + 2, 0)
            compute(lo + c0 + 1, 1, False)
            return carry

        lax.fori_loop(0, (NROWS // 16 - 1) // 2, pair, 0)
        compute(lo + NROWS // 16 - 1, 0, False)
        pltpu.make_async_copy(sbuf.at[0], acc.at[dsciv.at[0]], sems).wait()
        plsc.subcore_barrier()
        pltpu.sync_copy(acc.at[pl.ds(tid * NSLICE, NSLICE)],
                        outslice.at[pl.ds(tid * NSLICE, NSLICE)])

    @pl.when(cid == 0)
    def _():
        run(t0_hbm, out_hbm.at[0])

    @pl.when(cid == 1)
    def _():
        run(t1_hbm, out_hbm.at[1])


def _lrelu(x, s):
    return jnp.where(x > 0, x, s * x)


_RB = N // 16


def _mm_body(x_ref, w_ref, b_ref, o_ref):
    o_ref[...] = jnp.dot(x_ref[...], w_ref[...],
                         preferred_element_type=_f32) + b_ref[...]


def _mm(x, W, b):
    K = x.shape[1]
    return pl.pallas_call(
        _mm_body,
        grid=(16,),
        in_specs=[pl.BlockSpec((_RB, K), lambda i: (i, 0)),
                  pl.BlockSpec((K, 128), lambda i: (0, 0)),
                  pl.BlockSpec((1, 128), lambda i: (0, 0))],
        out_specs=pl.BlockSpec((_RB, 128), lambda i: (i, 0)),
        out_shape=jax.ShapeDtypeStruct((N, 128), _f32),
    )(x, W, b.reshape(1, 128))


def _ew1_body(o0_ref, o1_ref, b_ref, h_ref, ps_ref):
    h = jnp.concatenate([o0_ref[...], o1_ref[...]], axis=1) + b_ref[...]
    h = _lrelu(h, 0.01)
    h_ref[...] = h
    lane = lax.broadcasted_iota(_i32, (1, 1, 128), 2)
    ps_ref[...] = jnp.where(lane == 0, jnp.sum(h),
                            jnp.where(lane == 1, jnp.sum(h * h), 0.0))


def _ew1(o0, o1, bias):
    return pl.pallas_call(
        _ew1_body,
        grid=(16,),
        in_specs=[pl.BlockSpec((_RB, 32), lambda i: (i, 0)),
                  pl.BlockSpec((_RB, 32), lambda i: (i, 0)),
                  pl.BlockSpec((1, HID), lambda i: (0, 0))],
        out_specs=(pl.BlockSpec((_RB, HID), lambda i: (i, 0)),
                   pl.BlockSpec((1, 1, 128), lambda i: (i, 0, 0))),
        out_shape=(jax.ShapeDtypeStruct((N, HID), _f32),
                   jax.ShapeDtypeStruct((16, 1, 128), _f32)),
    )(o0, o1, bias.reshape(1, HID))


def _ew2_body(o0_ref, o1_ref, xp_ref, wr_ref, b_ref, h_ref):
    agg = jnp.concatenate([o0_ref[...], o1_ref[...]], axis=1)
    h = agg + wr_ref[...] * xp_ref[...] + b_ref[...]
    h_ref[...] = jax.nn.relu(h)


def _ew2(o0, o1, xp, wr, bias):
    return pl.pallas_call(
        _ew2_body,
        grid=(16,),
        in_specs=[pl.BlockSpec((_RB, 32), lambda i: (i, 0)),
                  pl.BlockSpec((_RB, 32), lambda i: (i, 0)),
                  pl.BlockSpec((_RB, HID), lambda i: (i, 0)),
                  pl.BlockSpec((_RB, 1), lambda i: (i, 0)),
                  pl.BlockSpec((1, HID), lambda i: (0, 0))],
        out_specs=pl.BlockSpec((_RB, HID), lambda i: (i, 0)),
        out_shape=jax.ShapeDtypeStruct((N, HID), _f32),
    )(o0, o1, xp, wr.reshape(N, 1), bias.reshape(1, HID))


def _mlp_body(f_ref, w1_ref, b1_ref, w2_ref, b2_ref, o_ref):
    z = jnp.dot(f_ref[...], w1_ref[...], preferred_element_type=_f32) + b1_ref[...]
    z = _lrelu(z, 0.01)
    z = jnp.dot(z, w2_ref[...], preferred_element_type=_f32) + b2_ref[...]
    o_ref[...] = jax.nn.sigmoid(z)


def _mlp(feats, W1, b1, W2p, b2):
    return pl.pallas_call(
        _mlp_body,
        out_shape=jax.ShapeDtypeStruct((391, 128), _f32),
    )(feats, W1, b1.reshape(1, 128), W2p, b2.reshape(1, 1))


def kernel(x, edge_index, batch, W_l, b_l, W_r, b_r, att_v2, bias_v2, gamma,
           beta, Wg1, att_src1, att_dst1, bias_g1, Wg2, att_src2, att_dst2,
           bias_g2, W1, b1, W2, b2):
    src2d = edge_index[0].reshape(NROWS, 128)
    dst2d = edge_index[1].reshape(NROWS, 128)
    zero1 = jnp.zeros((N,), _f32)
    zero2 = jnp.zeros((N, 32), _f32)

    Wlr = jnp.concatenate([W_l, W_r], axis=1)
    blr = jnp.concatenate([b_l, b_r])
    xlr = _mm(x, Wlr, blr)
    xl = xlr[:, :HID]
    xr = xlr[:, HID:]
    att_pad = jnp.concatenate([att_v2, jnp.zeros((64,), _f32)])
    src2d64 = edge_index[0].reshape(NR64, 64)
    dst2d64 = edge_index[1].reshape(NR64, 64)
    w1, s1p = _k1(xl, xr, att_pad, src2d64, dst2d64, zero1)
    s1 = s1p.sum(axis=0)
    r1 = 1.0 / (s1 + 1e-16)
    a1 = _k4(dst2d64, w1, r1).reshape(NROWS, 128)
    o1 = _k2(xl[:, :32], xl[:, 32:64], src2d, dst2d, a1, zero2)
    h, ps = _ew1(o1[0], o1[1], bias_v2)

    cnt = jnp.float32(N * HID)
    mean = jnp.sum(ps[:, 0, 0]) / cnt
    var = jnp.sum(ps[:, 0, 1]) / cnt - mean * mean
    inv = 1.0 / (jnp.sqrt(var) + 1e-5)
    a_vec = inv * gamma
    b_vec = beta - mean * inv * gamma

    for li, (W, a_s, a_d, bias) in enumerate((
            (Wg1, att_src1, att_dst1, bias_g1),
            (Wg2, att_src2, att_dst2, bias_g2))):
        if li == 0:
            W_eff = a_vec[:, None] * W
            b_eff = b_vec @ W
        else:
            W_eff = W
            b_eff = jnp.zeros((HID,), _f32)
        Waug = jnp.concatenate(
            [W_eff, (W_eff @ a_s)[:, None], (W_eff @ a_d)[:, None],
             jnp.zeros((HID, 62), _f32)], axis=1)
        baug = jnp.concatenate(
            [b_eff, (b_eff @ a_s)[None], (b_eff @ a_d)[None],
             jnp.zeros((62,), _f32)])
        paug = _mm(h, Waug, baug)
        xp = paug[:, :HID]
        asv = paug[:, HID]
        adv = paug[:, HID + 1]
        w_e = _k3(asv, adv, src2d64, dst2d64)
        sp = _ks(dst2d64, w_e, zero1)
        wself = jnp.exp(_lrelu(asv + adv, 0.2))
        s = sp.sum(axis=0) + wself
        r = 1.0 / (s + 1e-16)
        a_e = _k4(dst2d64, w_e, r).reshape(NROWS, 128)
        o = _k2(xp[:, :32], xp[:, 32:64], src2d, dst2d, a_e, zero2)
        h = _ew2(o[0], o[1], xp, wself * r, bias)

    feats = h.reshape(391, 128 * HID)
    W2p = jnp.pad(W2, ((0, 0), (0, 127)))
    z = _mlp(feats, W1, b1, W2p, b2)
    return z[:, :1]
